# baseline (speedup 1.0000x reference)
import functools

import jax
import jax.numpy as jnp
from jax import lax
from jax.experimental import pallas as pl
from jax.experimental.pallas import tpu as pltpu
from jax.experimental.pallas import tpu_sc as plsc

B = 16384
VOCAB = 100000
EMB = 64
NG = 19
NGR = 30
NC_CAT = 20
NCAT = NG + NGR + NC_CAT
CATP = 128
FD = 128
EPS = 1e-3

IDXW = 128
CW = 256
NCHUNK = 391
CPW_MAX = 13
OUTPAD = 2048
POSBITS = 14
POSMASK = (1 << POSBITS) - 1
BLK = 2048
SUBB = 512
NSUB = BLK // SUBB
NBLK = B // BLK


def _make_gather():
    info = plsc.get_sparse_core_info()
    num_cores, num_subcores = info.num_cores, info.num_subcores
    nw = num_cores * num_subcores
    bpw = B // nw
    mchunks = bpw // IDXW
    nvec = B // 16

    mesh = plsc.VectorSubcoreMesh(core_axis_name="c", subcore_axis_name="s")

    @functools.partial(
        pl.kernel,
        out_type=(
            jax.ShapeDtypeStruct((B + OUTPAD, 128), jnp.float32),
            jax.ShapeDtypeStruct((nw, 8, bpw), jnp.int32),
        ),
        mesh=mesh,
        compiler_params=pltpu.CompilerParams(needs_layout_passes=False),
        scratch_types=[
            pltpu.VMEM((B,), jnp.int32),
            pltpu.VMEM((B + 32,), jnp.int32),
            pltpu.VMEM((B + 32,), jnp.int32),
            pltpu.VMEM((2, 64, CW), jnp.float32),
            pltpu.VMEM((2, 128, 128), jnp.float32),
            pltpu.VMEM((2, 128), jnp.int32),
            pltpu.VMEM((8, bpw), jnp.int32),
            pltpu.SemaphoreType.DMA,
            pltpu.SemaphoreType.DMA,
            pltpu.SemaphoreType.DMA,
        ],
    )
    def gather(ids_hbm, table_t, gmap, grmap, cmap,
               emb_out, cats_out,
               ids_v, ml_pk, cl_pk,
               chunk_v, stage_v, pidx_v, cats_v,
               sem_m, sem_c, sem_s):
        wid = lax.axis_index("s") * num_cores + lax.axis_index("c")
        base = wid * bpw
        iota = lax.iota(jnp.int32, 16)
        one = jnp.full((16,), 1, jnp.int32)

        pltpu.sync_copy(ids_hbm, ids_v)

        map_cps = []
        for mc in range(mchunks):
            sl = pl.ds(base + mc * IDXW, IDXW)
            dsl = pl.ds(mc * IDXW, IDXW)
            map_cps.append(pltpu.async_copy(
                gmap.at[ids_v.at[sl]], cats_v.at[0, dsl], sem_m))
            map_cps.append(pltpu.async_copy(
                grmap.at[ids_v.at[sl]], cats_v.at[1, dsl], sem_m))
            map_cps.append(pltpu.async_copy(
                cmap.at[ids_v.at[sl]], cats_v.at[2, dsl], sem_m))

        c0 = (wid * NCHUNK) // nw
        c1 = ((wid + 1) * NCHUNK) // nw
        lo = c0 * CW
        hi = c1 * CW
        trash = jnp.full((16,), B + 16, jnp.int32)

        @plsc.parallel_loop(0, nvec, unroll=4, carry=jnp.int32(0))
        def prefilter(i, cnt):
            vec = ids_v[pl.ds(i * 16, 16)]
            pos = jnp.full((16,), 16, jnp.int32) * i + iota
            m = (vec >= lo) & (vec < hi)
            incl = plsc.cumsum(jnp.where(m, 1, 0))
            tgt = jnp.where(m, one * cnt + incl - 1, trash)
            pk = lax.shift_left(vec, POSBITS) | pos
            plsc.store_scatter(ml_pk, [tgt], pk)
            return cnt + incl[15]

        cnt = prefilter
        ml_pk[pl.ds(cnt, 16)] = jnp.full((16,), -1, jnp.int32)

        def fire_chunk(k):
            start = pl.multiple_of((c0 + k) * CW, CW)
            return pltpu.async_copy(
                table_t.at[:, pl.ds(start, CW)], chunk_v.at[k % 2], sem_c)

        @pl.when(c0 < c1)
        def _():
            fire_chunk(0)

        trip = (cnt + 15) >> 4
        bsent = jnp.full((16,), B, jnp.int32)
        g16 = jnp.int32(0)

        for k in range(CPW_MAX):
            ck = c0 + k
            active = ck < c1

            @pl.when(active)
            def _(k=k):
                pltpu.make_async_copy(
                    table_t.at[:, pl.ds(pl.multiple_of((c0 + k) * CW, CW), CW)],
                    chunk_v.at[k % 2], sem_c).wait()

            @pl.when(c0 + k + 1 < c1)
            def _(k=k):
                fire_chunk(k + 1)

            @plsc.parallel_loop(0, trip, unroll=4, carry=jnp.int32(0))
            def cscan(i, nc, ck=ck):
                pk = ml_pk[pl.ds(i * 16, 16)]
                vid = lax.shift_right_arithmetic(pk, POSBITS)
                m = lax.shift_right_arithmetic(vid, 8) == ck
                incl = plsc.cumsum(jnp.where(m, 1, 0))
                tgt = jnp.where(m, one * nc + incl - 1, trash)
                plsc.store_scatter(cl_pk, [tgt], pk)
                return nc + incl[15]

            nc = cscan
            cl_pk[pl.ds(nc, 16)] = jnp.full((16,), -1, jnp.int32)

            def egroup(g, g16, k=k, ck=ck):
                slot = (g16 >> 3) & 1
                slot_vec = one * slot

                @pl.when((g16 & 7) == 0)
                def _():
                    for q in range(8):
                        plsc.store_scatter(
                            pidx_v, [slot_vec, iota + 16 * q], bsent)

                pk = cl_pk[pl.ds(g * 16, 16)]
                vid = lax.shift_right_arithmetic(pk, POSBITS)
                pos = pk & POSMASK
                spos = jnp.where(pk < 0, bsent, pos)
                l_raw = vid - ck * CW
                l_vec = jnp.minimum(jnp.maximum(l_raw, 0), CW - 1)
                rows = iota + (g16 & 7) * 16
                kvec = one * (k % 2)

                @plsc.parallel_loop(0, EMB, unroll=8)
                def _(f, kvec=kvec, l_vec=l_vec, slot_vec=slot_vec,
                      rows=rows):
                    fvec = one * f
                    feats = plsc.load_gather(chunk_v, [kvec, fvec, l_vec])
                    plsc.store_scatter(stage_v, [slot_vec, rows, fvec],
                                       feats)

                plsc.store_scatter(pidx_v, [slot_vec, rows], spos)

                @pl.when((g16 & 7) == 7)
                def _():
                    pltpu.async_copy(stage_v.at[(g16 >> 3) & 1],
                                     emb_out.at[pidx_v.at[(g16 >> 3) & 1]],
                                     sem_s)

                @pl.when(((g16 & 7) == 7) & ((g16 >> 3) >= 1))
                def _():
                    pltpu.make_async_copy(
                        emb_out.at[pl.ds(0, 128)], stage_v.at[0],
                        sem_s).wait()

                return g16 + 1

            ngrp = (nc + 15) >> 4
            g16 = lax.fori_loop(0, ngrp, egroup, g16)

        @pl.when((g16 & 7) != 0)
        def _():
            pltpu.async_copy(stage_v.at[(g16 >> 3) & 1],
                             emb_out.at[pidx_v.at[(g16 >> 3) & 1]], sem_s)

        full_fires = g16 >> 3
        fired = full_fires + jnp.where((g16 & 7) != 0, 1, 0)
        waited = jnp.maximum(full_fires - 1, 0)

        def drain(i, carry):
            pltpu.make_async_copy(
                emb_out.at[pl.ds(0, 128)], stage_v.at[0], sem_s).wait()
            return carry

        lax.fori_loop(0, fired - waited, drain, jnp.int32(0))

        for cp in map_cps:
            cp.wait()
        pltpu.sync_copy(cats_v, cats_out.at[wid])

    return gather


def _tc_body(cats_ref, emb_ref, we_ref, wc_ref,
             ge_ref, be_ref, me_ref, ve_ref,
             gc_ref, bc_ref, mc_ref, vc_ref, out_ref):
    g = jnp.concatenate([cats_ref[k, 0:1, :] for k in range(NSUB)], axis=1)
    gr = jnp.concatenate([cats_ref[k, 1:2, :] for k in range(NSUB)], axis=1)
    c = jnp.concatenate([cats_ref[k, 2:3, :] for k in range(NSUB)], axis=1)

    sub = lax.broadcasted_iota(jnp.int32, (CATP, BLK), 0)
    hot = (sub == g) | (sub == gr + NG) | (sub == c + (NG + NGR))

    s_cat = gc_ref[...] * lax.rsqrt(vc_ref[...] + EPS)
    t_cat = bc_ref[...] - mc_ref[...] * s_cat
    xcat_t = jnp.where(hot, s_cat + t_cat, t_cat)

    s_emb = ge_ref[...] * lax.rsqrt(ve_ref[...] + EPS)
    t_emb = be_ref[...] - me_ref[...] * s_emb
    lane = lax.broadcasted_iota(jnp.int32, (BLK, 128), 1)
    xemb = jnp.where(lane < EMB,
                     emb_ref[...] * s_emb + t_emb,
                     jnp.float32(0.0))

    acc = lax.dot_general(xemb, we_ref[...], (((1,), (0,)), ((), ())),
                          preferred_element_type=jnp.float32)
    acc = acc + lax.dot_general(xcat_t, wc_ref[...], (((0,), (0,)), ((), ())),
                                preferred_element_type=jnp.float32)
    out_ref[...] = acc


def _const2(i):
    return (0, 0)


_tc_call = pl.pallas_call(
    _tc_body,
    grid=(NBLK,),
    in_specs=[
        pl.BlockSpec((NSUB, 8, SUBB), lambda i: (i, 0, 0)),
        pl.BlockSpec((BLK, 128), lambda i: (i, 0)),
        pl.BlockSpec((128, FD), _const2),
        pl.BlockSpec((CATP, FD), _const2),
        pl.BlockSpec((1, 128), _const2),
        pl.BlockSpec((1, 128), _const2),
        pl.BlockSpec((1, 128), _const2),
        pl.BlockSpec((1, 128), _const2),
        pl.BlockSpec((CATP, 1), _const2),
        pl.BlockSpec((CATP, 1), _const2),
        pl.BlockSpec((CATP, 1), _const2),
        pl.BlockSpec((CATP, 1), _const2),
    ],
    out_specs=pl.BlockSpec((BLK, FD), lambda i: (i, 0)),
    out_shape=jax.ShapeDtypeStruct((B, FD), jnp.float32),
)


def kernel(article_id, emb_table, group_map, graphical_map, colour_map,
           gamma, beta, moving_mean, moving_var, W):
    emb_rows, cats = _make_gather()(
        article_id, emb_table.T, group_map, graphical_map, colour_map)

    pad = CATP - NCAT
    epad = 128 - EMB
    we = jnp.pad(W[:EMB], ((0, epad), (0, 0)))
    wc = jnp.pad(W[EMB:], ((0, pad), (0, 0)))
    ge = jnp.pad(gamma[:EMB], (0, epad)).reshape(1, 128)
    be = jnp.pad(beta[:EMB], (0, epad)).reshape(1, 128)
    me = jnp.pad(moving_mean[:EMB], (0, epad)).reshape(1, 128)
    ve = jnp.pad(moving_var[:EMB], (0, epad),
                 constant_values=1.0).reshape(1, 128)
    gc = jnp.pad(gamma[EMB:], (0, pad)).reshape(CATP, 1)
    bc = jnp.pad(beta[EMB:], (0, pad)).reshape(CATP, 1)
    mc = jnp.pad(moving_mean[EMB:], (0, pad)).reshape(CATP, 1)
    vc = jnp.pad(moving_var[EMB:], (0, pad),
                 constant_values=1.0).reshape(CATP, 1)

    return _tc_call(cats, emb_rows, we, wc, ge, be, me, ve, gc, bc, mc, vc)


# speedup vs baseline: 3.2397x; 3.2397x over previous
import functools

import jax
import jax.numpy as jnp
from jax import lax
from jax.experimental import pallas as pl
from jax.experimental.pallas import tpu as pltpu
from jax.experimental.pallas import tpu_sc as plsc

B = 16384
VOCAB = 100000
EMB = 64
NG = 19
NGR = 30
NC_CAT = 20
NCAT = NG + NGR + NC_CAT
CATP = 128
FD = 128
EPS = 1e-3

IDXW = 128
GRP = 16
BLK = 2048
SUBB = 512
NSUB = BLK // SUBB
NBLK = B // BLK


def _make_gather():
    info = plsc.get_sparse_core_info()
    num_cores, num_subcores = info.num_cores, info.num_subcores
    nw = num_cores * num_subcores
    bpw = B // nw
    chunks = bpw // IDXW
    ngrp = bpw // GRP

    mesh = plsc.VectorSubcoreMesh(core_axis_name="c", subcore_axis_name="s")

    @functools.partial(
        pl.kernel,
        out_type=(
            jax.ShapeDtypeStruct((B, EMB), jnp.float32),
            jax.ShapeDtypeStruct((nw, 8, bpw), jnp.int32),
        ),
        mesh=mesh,
        scratch_types=[
            pltpu.VMEM((bpw,), jnp.int32),
            pltpu.VMEM((bpw, EMB), jnp.float32),
            pltpu.VMEM((8, bpw), jnp.int32),
            pltpu.SemaphoreType.DMA,
            pltpu.SemaphoreType.DMA,
        ],
    )
    def gather(ids_hbm, emb_hbm, gmap, grmap, cmap,
               emb_out, cats_out,
               idx_v, rows_v, cats_v, sem, sem2):
        wid = lax.axis_index("s") * num_cores + lax.axis_index("c")
        base = wid * bpw

        pltpu.sync_copy(ids_hbm.at[pl.ds(base, bpw)], idx_v)

        map_cps = []
        for c in range(chunks):
            sl = pl.ds(c * IDXW, IDXW)
            map_cps.append(pltpu.async_copy(
                gmap.at[idx_v.at[sl]], cats_v.at[0, sl], sem))
            map_cps.append(pltpu.async_copy(
                grmap.at[idx_v.at[sl]], cats_v.at[1, sl], sem))
            map_cps.append(pltpu.async_copy(
                cmap.at[idx_v.at[sl]], cats_v.at[2, sl], sem))

        def body(g, carry):
            vec = idx_v[pl.ds(g * GRP, GRP)]
            grp_cps = []
            for jj in range(GRP):
                v = vec[jj]
                grp_cps.append(pltpu.async_copy(
                    emb_hbm.at[pl.ds(v, 1)],
                    rows_v.at[pl.ds(g * GRP + jj, 1)], sem2))

            @pl.when(g > 1)
            def _():
                for cp in grp_cps:
                    cp.wait()

            return carry

        lax.fori_loop(0, ngrp, body, 0)

        for jj in range(2 * GRP):
            pltpu.make_async_copy(
                emb_hbm.at[pl.ds(0, 1)],
                rows_v.at[pl.ds(jj % GRP, 1)], sem2).wait()
        for cp in map_cps:
            cp.wait()

        pltpu.sync_copy(rows_v, emb_out.at[pl.ds(base, bpw)])
        pltpu.sync_copy(cats_v, cats_out.at[wid])

    return gather


def _tc_body(cats_ref, emb_ref, we_ref, wc_ref,
             ge_ref, be_ref, me_ref, ve_ref,
             gc_ref, bc_ref, mc_ref, vc_ref, out_ref):
    g = jnp.concatenate([cats_ref[k, 0:1, :] for k in range(NSUB)], axis=1)
    gr = jnp.concatenate([cats_ref[k, 1:2, :] for k in range(NSUB)], axis=1)
    c = jnp.concatenate([cats_ref[k, 2:3, :] for k in range(NSUB)], axis=1)

    sub = lax.broadcasted_iota(jnp.int32, (CATP, BLK), 0)
    hot = (sub == g) | (sub == gr + NG) | (sub == c + (NG + NGR))

    s_cat = gc_ref[...] * lax.rsqrt(vc_ref[...] + EPS)
    t_cat = bc_ref[...] - mc_ref[...] * s_cat
    xcat_t = jnp.where(hot, s_cat + t_cat, t_cat)

    s_emb = ge_ref[...] * lax.rsqrt(ve_ref[...] + EPS)
    t_emb = be_ref[...] - me_ref[...] * s_emb
    xemb = emb_ref[...] * s_emb + t_emb

    acc = lax.dot_general(xemb, we_ref[...], (((1,), (0,)), ((), ())),
                          preferred_element_type=jnp.float32)
    acc = acc + lax.dot_general(xcat_t, wc_ref[...], (((0,), (0,)), ((), ())),
                                preferred_element_type=jnp.float32)
    out_ref[...] = acc


def _const2(i):
    return (0, 0)


_tc_call = pl.pallas_call(
    _tc_body,
    grid=(NBLK,),
    in_specs=[
        pl.BlockSpec((NSUB, 8, SUBB), lambda i: (i, 0, 0)),
        pl.BlockSpec((BLK, EMB), lambda i: (i, 0)),
        pl.BlockSpec((EMB, FD), _const2),
        pl.BlockSpec((CATP, FD), _const2),
        pl.BlockSpec((1, EMB), _const2),
        pl.BlockSpec((1, EMB), _const2),
        pl.BlockSpec((1, EMB), _const2),
        pl.BlockSpec((1, EMB), _const2),
        pl.BlockSpec((CATP, 1), _const2),
        pl.BlockSpec((CATP, 1), _const2),
        pl.BlockSpec((CATP, 1), _const2),
        pl.BlockSpec((CATP, 1), _const2),
    ],
    out_specs=pl.BlockSpec((BLK, FD), lambda i: (i, 0)),
    out_shape=jax.ShapeDtypeStruct((B, FD), jnp.float32),
)


def kernel(article_id, emb_table, group_map, graphical_map, colour_map,
           gamma, beta, moving_mean, moving_var, W):
    emb_rows, cats = _make_gather()(
        article_id, emb_table, group_map, graphical_map, colour_map)

    pad = CATP - NCAT
    we = W[:EMB]
    wc = jnp.pad(W[EMB:], ((0, pad), (0, 0)))
    ge = gamma[:EMB].reshape(1, EMB)
    be = beta[:EMB].reshape(1, EMB)
    me = moving_mean[:EMB].reshape(1, EMB)
    ve = moving_var[:EMB].reshape(1, EMB)
    gc = jnp.pad(gamma[EMB:], (0, pad)).reshape(CATP, 1)
    bc = jnp.pad(beta[EMB:], (0, pad)).reshape(CATP, 1)
    mc = jnp.pad(moving_mean[EMB:], (0, pad)).reshape(CATP, 1)
    vc = jnp.pad(moving_var[EMB:], (0, pad), constant_values=1.0).reshape(CATP, 1)

    return _tc_call(cats, emb_rows, we, wc, ge, be, me, ve, gc, bc, mc, vc)


# speedup vs baseline: 3.2997x; 1.0185x over previous
import functools

import jax
import jax.numpy as jnp
from jax import lax
from jax.experimental import pallas as pl
from jax.experimental.pallas import tpu as pltpu
from jax.experimental.pallas import tpu_sc as plsc

B = 16384
VOCAB = 100000
EMB = 64
NG = 19
NGR = 30
NC_CAT = 20
NCAT = NG + NGR + NC_CAT
CATP = 128
FD = 128
EPS = 1e-3

IDXW = 128
GRP = 16
BLK = 2048
SUBB = 512
NSUB = BLK // SUBB
NBLK = B // BLK


def _make_gather():
    info = plsc.get_sparse_core_info()
    num_cores, num_subcores = info.num_cores, info.num_subcores
    nw = num_cores * num_subcores
    bpw = B // nw
    chunks = bpw // IDXW
    ngrp = bpw // GRP

    mesh = plsc.VectorSubcoreMesh(core_axis_name="c", subcore_axis_name="s")

    @functools.partial(
        pl.kernel,
        out_type=(
            jax.ShapeDtypeStruct((B, EMB), jnp.float32),
            jax.ShapeDtypeStruct((nw, 8, bpw), jnp.int32),
        ),
        mesh=mesh,
        scratch_types=[
            pltpu.VMEM((bpw,), jnp.int32),
            pltpu.VMEM((bpw, EMB), jnp.float32),
            pltpu.VMEM((8, bpw), jnp.int32),
            pltpu.SemaphoreType.DMA,
            pltpu.SemaphoreType.DMA,
        ],
    )
    def gather(ids_hbm, emb_hbm, gmap, grmap, cmap,
               emb_out, cats_out,
               idx_v, rows_v, cats_v, sem, sem2):
        wid = lax.axis_index("s") * num_cores + lax.axis_index("c")
        base = wid * bpw

        pltpu.sync_copy(ids_hbm.at[pl.ds(base, bpw)], idx_v)

        map_cps = []
        for c in range(chunks):
            sl = pl.ds(c * IDXW, IDXW)
            map_cps.append(pltpu.async_copy(
                gmap.at[idx_v.at[sl]], cats_v.at[0, sl], sem))
            map_cps.append(pltpu.async_copy(
                grmap.at[idx_v.at[sl]], cats_v.at[1, sl], sem))
            map_cps.append(pltpu.async_copy(
                cmap.at[idx_v.at[sl]], cats_v.at[2, sl], sem))

        def body(g, carry):
            vec = idx_v[pl.ds(g * GRP, GRP)]
            grp_cps = []
            for jj in range(GRP):
                v = vec[jj]
                grp_cps.append(pltpu.async_copy(
                    emb_hbm.at[pl.ds(v, 1)],
                    rows_v.at[pl.ds(g * GRP + jj, 1)], sem2))

            @pl.when(g > 2)
            def _():
                for cp in grp_cps:
                    cp.wait()

            return carry

        lax.fori_loop(0, ngrp, body, 0)

        for jj in range(3 * GRP):
            pltpu.make_async_copy(
                emb_hbm.at[pl.ds(0, 1)],
                rows_v.at[pl.ds(jj % GRP, 1)], sem2).wait()
        for cp in map_cps:
            cp.wait()

        pltpu.sync_copy(rows_v, emb_out.at[pl.ds(base, bpw)])
        pltpu.sync_copy(cats_v, cats_out.at[wid])

    return gather


def _tc_body(cats_ref, emb_ref, we_ref, wc_ref,
             ge_ref, be_ref, me_ref, ve_ref,
             gc_ref, bc_ref, mc_ref, vc_ref, out_ref):
    g = jnp.concatenate([cats_ref[k, 0:1, :] for k in range(NSUB)], axis=1)
    gr = jnp.concatenate([cats_ref[k, 1:2, :] for k in range(NSUB)], axis=1)
    c = jnp.concatenate([cats_ref[k, 2:3, :] for k in range(NSUB)], axis=1)

    sub = lax.broadcasted_iota(jnp.int32, (CATP, BLK), 0)
    hot = (sub == g) | (sub == gr + NG) | (sub == c + (NG + NGR))

    s_cat = gc_ref[...] * lax.rsqrt(vc_ref[...] + EPS)
    t_cat = bc_ref[...] - mc_ref[...] * s_cat
    xcat_t = jnp.where(hot, s_cat + t_cat, t_cat)

    s_emb = ge_ref[...] * lax.rsqrt(ve_ref[...] + EPS)
    t_emb = be_ref[...] - me_ref[...] * s_emb
    xemb = emb_ref[...] * s_emb + t_emb

    acc = lax.dot_general(xemb, we_ref[...], (((1,), (0,)), ((), ())),
                          preferred_element_type=jnp.float32)
    acc = acc + lax.dot_general(xcat_t, wc_ref[...], (((0,), (0,)), ((), ())),
                                preferred_element_type=jnp.float32)
    out_ref[...] = acc


def _const2(i):
    return (0, 0)


_tc_call = pl.pallas_call(
    _tc_body,
    grid=(NBLK,),
    in_specs=[
        pl.BlockSpec((NSUB, 8, SUBB), lambda i: (i, 0, 0)),
        pl.BlockSpec((BLK, EMB), lambda i: (i, 0)),
        pl.BlockSpec((EMB, FD), _const2),
        pl.BlockSpec((CATP, FD), _const2),
        pl.BlockSpec((1, EMB), _const2),
        pl.BlockSpec((1, EMB), _const2),
        pl.BlockSpec((1, EMB), _const2),
        pl.BlockSpec((1, EMB), _const2),
        pl.BlockSpec((CATP, 1), _const2),
        pl.BlockSpec((CATP, 1), _const2),
        pl.BlockSpec((CATP, 1), _const2),
        pl.BlockSpec((CATP, 1), _const2),
    ],
    out_specs=pl.BlockSpec((BLK, FD), lambda i: (i, 0)),
    out_shape=jax.ShapeDtypeStruct((B, FD), jnp.float32),
)


def kernel(article_id, emb_table, group_map, graphical_map, colour_map,
           gamma, beta, moving_mean, moving_var, W):
    emb_rows, cats = _make_gather()(
        article_id, emb_table, group_map, graphical_map, colour_map)

    pad = CATP - NCAT
    we = W[:EMB]
    wc = jnp.pad(W[EMB:], ((0, pad), (0, 0)))
    ge = gamma[:EMB].reshape(1, EMB)
    be = beta[:EMB].reshape(1, EMB)
    me = moving_mean[:EMB].reshape(1, EMB)
    ve = moving_var[:EMB].reshape(1, EMB)
    gc = jnp.pad(gamma[EMB:], (0, pad)).reshape(CATP, 1)
    bc = jnp.pad(beta[EMB:], (0, pad)).reshape(CATP, 1)
    mc = jnp.pad(moving_mean[EMB:], (0, pad)).reshape(CATP, 1)
    vc = jnp.pad(moving_var[EMB:], (0, pad), constant_values=1.0).reshape(CATP, 1)

    return _tc_call(cats, emb_rows, we, wc, ge, be, me, ve, gc, bc, mc, vc)
